# Initial kernel scaffold; baseline (speedup 1.0000x reference)
import jax
import jax.numpy as jnp
from jax.experimental import pallas as pl


def kernel(input, target):
    raise NotImplementedError("write your pallas kernel here")


# speedup vs baseline: 97.7045x; 97.7045x over previous
import jax
import jax.numpy as jnp
from jax import lax
from jax.experimental import pallas as pl

B = 512
D = 128
KPOS = 3
KNEG = 5
MARGIN = 1.0
BOUNDARY = 5.0
EPS_POS = 1e-07


def _loss_kernel(x_ref, tcol_ref, trow_ref, out_ref):
    x = x_ref[...]
    tcol = tcol_ref[...]
    trow = trow_ref[...]

    dot = lax.dot_general(x, x, (((1,), (1,)), ((), ())),
                          preferred_element_type=jnp.float32)

    rows = lax.broadcasted_iota(jnp.int32, (B, B), 0)
    cols = lax.broadcasted_iota(jnp.int32, (B, B), 1)
    eye = rows == cols

    diag_m = jnp.where(eye, dot, 0.0)
    sq_col = jnp.sum(diag_m, axis=1, keepdims=True)
    sq_row = jnp.sum(diag_m, axis=0, keepdims=True)

    d = sq_row - 2.0 * dot + sq_col
    d = jnp.maximum(d, 0.0)
    dist = jnp.where(d == 0.0, 0.0, jnp.sqrt(d))

    same = tcol == trow

    counts = jnp.sum(same.astype(jnp.float32), axis=1, keepdims=True)
    anchors = counts < BOUNDARY

    inf = jnp.float32(jnp.inf)

    vn = jnp.where(same, inf, dist)
    cur = vn
    thresh = None
    for j in range(KNEG):
        thresh = jnp.min(cur, axis=0, keepdims=True)
        if j < KNEG - 1:
            hit = jnp.where(cur == thresh, rows, B)
            first = jnp.min(hit, axis=0, keepdims=True)
            cur = jnp.where(rows == first, inf, cur)
    mn = (~same) & (vn <= thresh)

    mpv = jnp.where(same & ~eye, dist, -inf)
    w_mask = mn & anchors
    wf = w_mask.astype(jnp.float32)
    s_total = jnp.float32(0.0)
    n_total = jnp.float32(0.0)
    curp = mpv
    for k in range(KPOS):
        pd = jnp.max(curp, axis=1, keepdims=True)
        if k < KPOS - 1:
            hit = jnp.where(curp == pd, cols, B)
            first = jnp.min(hit, axis=1, keepdims=True)
            curp = jnp.where(cols == first, -inf, curp)
        valid = (pd > -inf).astype(jnp.float32)
        t = jnp.maximum(jnp.where(pd > -inf, pd, 0.0) + MARGIN - dist, 0.0)
        wk = wf * valid
        s_total = s_total + jnp.sum(wk * t)
        n_total = n_total + jnp.sum(wk * (t > EPS_POS).astype(jnp.float32))

    out_ref[...] = (s_total / (n_total + EPS_POS)).reshape(1, 1)


def kernel(input, target):
    tcol = target.reshape(B, 1)
    trow = target.reshape(1, B)
    out = pl.pallas_call(
        _loss_kernel,
        out_shape=jax.ShapeDtypeStruct((1, 1), jnp.float32),
    )(input, tcol, trow)
    return out.reshape(())
